# baseline (device time: 128696 ns/iter reference)
import jax
import jax.numpy as jnp
from jax import lax
from jax.experimental import pallas as pl
from jax.experimental.pallas import tpu as pltpu

N_DEV = 16
HQ = 8
DH = 128
SQ = 256
D = HQ * DH
SCALE = 0.08838834764831843
ROWS = SQ // N_DEV
CW = D + 128


def kernel(x, Wq, Wo, K_ext, V_ext):
    skv = K_ext.shape[1]

    xb = x[0].astype(jnp.bfloat16)
    Wqb = Wq.astype(jnp.bfloat16)
    Wob = Wo.astype(jnp.bfloat16)
    Kb = jnp.transpose(K_ext[0], (1, 0, 2)).astype(jnp.bfloat16)
    Vb = jnp.transpose(V_ext[0], (1, 0, 2)).astype(jnp.bfloat16)

    def body(x_ref, wq_ref, wo_ref, k_ref, v_ref, out_ref,
             acc_ref, recv_ref, send_sems, recv_sems, credit_sem):
        my = lax.axis_index("i")
        left = (my - 1) % N_DEV
        right = (my + 1) % N_DEV

        barrier = pltpu.get_barrier_semaphore()
        for nbr in (left, right):
            pl.semaphore_signal(barrier, inc=1, device_id=(nbr,),
                                device_id_type=pl.DeviceIdType.MESH)
        pl.semaphore_wait(barrier, 2)

        q = lax.dot_general(x_ref[...], wq_ref[...],
                            (((1,), (0,)), ((), ())),
                            preferred_element_type=jnp.float32)
        q = (q * SCALE).astype(jnp.bfloat16)

        ls = []
        for h in range(HQ):
            qh = q[:, h * DH:(h + 1) * DH]
            s = lax.dot_general(qh, k_ref[h],
                                (((1,), (1,)), ((), ())),
                                preferred_element_type=jnp.float32)
            p = jnp.exp(s)
            ls.append(jnp.sum(p, axis=1, keepdims=True))
            acc_h = lax.dot_general(p.astype(jnp.bfloat16), v_ref[h],
                                    (((1,), (0,)), ((), ())),
                                    preferred_element_type=jnp.float32)
            acc_ref[:, h * DH:(h + 1) * DH] = acc_h
        lblock = jnp.concatenate(
            ls + [jnp.zeros((SQ, 128 - HQ), jnp.float32)], axis=1)
        acc_ref[:, D:] = lblock

        n_hops = 2 * (N_DEV - 1)
        for t in range(n_hops):
            slot = t % 2
            if t >= 2:
                pl.semaphore_wait(credit_sem, 1)
            if t < N_DEV - 1:
                sc = (my - t) % N_DEV
                rc = (my - t - 1) % N_DEV
            else:
                ta = t - (N_DEV - 1)
                sc = (my + 1 - ta) % N_DEV
                rc = (my - ta) % N_DEV
            rdma = pltpu.make_async_remote_copy(
                src_ref=acc_ref.at[pl.ds(sc * ROWS, ROWS)],
                dst_ref=recv_ref.at[slot],
                send_sem=send_sems.at[slot],
                recv_sem=recv_sems.at[slot],
                device_id=(right,),
                device_id_type=pl.DeviceIdType.MESH,
            )
            rdma.start()
            rdma.wait()
            if t < N_DEV - 1:
                acc_ref[pl.ds(rc * ROWS, ROWS), :] = (
                    acc_ref[pl.ds(rc * ROWS, ROWS), :] + recv_ref[slot])
            else:
                acc_ref[pl.ds(rc * ROWS, ROWS), :] = recv_ref[slot]
            pl.semaphore_signal(credit_sem, inc=1, device_id=(left,),
                                device_id_type=pl.DeviceIdType.MESH)
        pl.semaphore_wait(credit_sem, 2)

        linv = 1.0 / acc_ref[:, D:D + HQ]
        outs = []
        for h in range(HQ):
            o = acc_ref[:, h * DH:(h + 1) * DH] * linv[:, h:h + 1]
            outs.append(o.astype(jnp.bfloat16))
        o = jnp.concatenate(outs, axis=1)
        res = lax.dot_general(o, wo_ref[...],
                              (((1,), (0,)), ((), ())),
                              preferred_element_type=jnp.float32)
        out_ref[0] = res

    return pl.pallas_call(
        body,
        out_shape=jax.ShapeDtypeStruct((1, SQ, D), jnp.float32),
        in_specs=[pl.BlockSpec(memory_space=pltpu.VMEM)] * 5,
        out_specs=pl.BlockSpec(memory_space=pltpu.VMEM),
        scratch_shapes=[
            pltpu.VMEM((SQ, CW), jnp.float32),
            pltpu.VMEM((2, ROWS, CW), jnp.float32),
            pltpu.SemaphoreType.DMA((2,)),
            pltpu.SemaphoreType.DMA((2,)),
            pltpu.SemaphoreType.REGULAR,
        ],
        compiler_params=pltpu.CompilerParams(collective_id=0),
    )(xb, Wqb, Wob, Kb, Vb)


# device time: 71984 ns/iter; 1.7878x vs baseline; 1.7878x over previous
import jax
import jax.numpy as jnp
from jax import lax
from jax.experimental import pallas as pl
from jax.experimental.pallas import tpu as pltpu

N_DEV = 16
HQ = 8
DH = 128
SQ = 256
D = HQ * DH
SCALE = 0.08838834764831843
CW = D + 128

RS_HALF = (128, 64, 32, 16)
RS_OFF = (0, 128, 192, 224)
STAGE_ROWS = 240


def kernel(x, Wq, Wo, K_ext, V_ext):
    skv = K_ext.shape[1]

    xb = x[0].astype(jnp.bfloat16)
    Wqb = Wq.astype(jnp.bfloat16)
    Wob = Wo.astype(jnp.bfloat16)
    Kb = K_ext[0].reshape(skv, D).astype(jnp.bfloat16)
    Vb = V_ext[0].reshape(skv, D).astype(jnp.bfloat16)

    def body(x_ref, wq_ref, wo_ref, k_ref, v_ref, out_ref,
             acc_ref, stage_ref, q_ref, send_sems, recv_sems):
        my = lax.axis_index("i")

        barrier = pltpu.get_barrier_semaphore()
        for k in range(4):
            pl.semaphore_signal(barrier, inc=1, device_id=(my ^ (1 << k),),
                                device_id_type=pl.DeviceIdType.MESH)
        pl.semaphore_wait(barrier, 4)

        q = lax.dot_general(x_ref[...], wq_ref[...],
                            (((1,), (0,)), ((), ())),
                            preferred_element_type=jnp.float32)
        q_ref[...] = (q * SCALE).astype(jnp.bfloat16)

        def attn_half(lo):
            qrows = q_ref[pl.ds(lo, 128), :]
            ls = []
            for h in range(HQ):
                qh = qrows[:, h * DH:(h + 1) * DH]
                s = lax.dot_general(qh, k_ref[:, h * DH:(h + 1) * DH],
                                    (((1,), (1,)), ((), ())),
                                    preferred_element_type=jnp.float32)
                p = jnp.exp(s)
                ls.append(jnp.sum(p, axis=1, keepdims=True))
                acc_h = lax.dot_general(p.astype(jnp.bfloat16),
                                        v_ref[:, h * DH:(h + 1) * DH],
                                        (((1,), (0,)), ((), ())),
                                        preferred_element_type=jnp.float32)
                acc_ref[pl.ds(lo, 128), h * DH:(h + 1) * DH] = acc_h
            lblock = jnp.concatenate(
                ls + [jnp.zeros((128, 128 - HQ), jnp.float32)], axis=1)
            acc_ref[pl.ds(lo, 128), D:] = lblock

        def exchange(step, src_slice, dst_ref_sliced, partner):
            return pltpu.make_async_remote_copy(
                src_ref=src_slice,
                dst_ref=dst_ref_sliced,
                send_sem=send_sems.at[step],
                recv_sem=recv_sems.at[step],
                device_id=(partner,),
                device_id_type=pl.DeviceIdType.MESH,
            )

        b0 = my & 1
        keep_lo = b0 * 128
        send_lo = (1 - b0) * 128
        attn_half(send_lo)
        rdma0 = exchange(0, acc_ref.at[pl.ds(send_lo, 128)],
                         stage_ref.at[pl.ds(0, 128)], my ^ 1)
        rdma0.start()
        attn_half(keep_lo)
        rdma0.wait()
        acc_ref[pl.ds(keep_lo, 128), :] = (
            acc_ref[pl.ds(keep_lo, 128), :] + stage_ref[pl.ds(0, 128), :])

        cur_lo = keep_lo
        for k in (1, 2, 3):
            half = RS_HALF[k]
            bk = (my >> k) & 1
            klo = cur_lo + bk * half
            slo = cur_lo + (1 - bk) * half
            rdma = exchange(k, acc_ref.at[pl.ds(slo, half)],
                            stage_ref.at[pl.ds(RS_OFF[k], half)], my ^ (1 << k))
            rdma.start()
            rdma.wait()
            acc_ref[pl.ds(klo, half), :] = (
                acc_ref[pl.ds(klo, half), :]
                + stage_ref[pl.ds(RS_OFF[k], half), :])
            cur_lo = klo

        sz = 16
        for j, k in enumerate((3, 2, 1, 0)):
            bk = (my >> k) & 1
            parent_lo = cur_lo - bk * sz
            rdma = exchange(4 + j, acc_ref.at[pl.ds(cur_lo, sz)],
                            acc_ref.at[pl.ds(cur_lo, sz)], my ^ (1 << k))
            rdma.start()
            rdma.wait()
            cur_lo = parent_lo
            sz *= 2

        linv = 1.0 / acc_ref[:, D:D + HQ]
        outs = []
        for h in range(HQ):
            o = acc_ref[:, h * DH:(h + 1) * DH] * linv[:, h:h + 1]
            outs.append(o.astype(jnp.bfloat16))
        o = jnp.concatenate(outs, axis=1)
        res = lax.dot_general(o, wo_ref[...],
                              (((1,), (0,)), ((), ())),
                              preferred_element_type=jnp.float32)
        out_ref[0] = res

    return pl.pallas_call(
        body,
        out_shape=jax.ShapeDtypeStruct((1, SQ, D), jnp.float32),
        in_specs=[pl.BlockSpec(memory_space=pltpu.VMEM)] * 5,
        out_specs=pl.BlockSpec(memory_space=pltpu.VMEM),
        scratch_shapes=[
            pltpu.VMEM((SQ, CW), jnp.float32),
            pltpu.VMEM((STAGE_ROWS, CW), jnp.float32),
            pltpu.VMEM((SQ, D), jnp.bfloat16),
            pltpu.SemaphoreType.DMA((8,)),
            pltpu.SemaphoreType.DMA((8,)),
        ],
        compiler_params=pltpu.CompilerParams(collective_id=0),
    )(xb, Wqb, Wob, Kb, Vb)


# device time: 60529 ns/iter; 2.1262x vs baseline; 1.1892x over previous
import jax
import jax.numpy as jnp
from jax import lax
from jax.experimental import pallas as pl
from jax.experimental.pallas import tpu as pltpu

N_DEV = 16
HQ = 8
DH = 128
SQ = 256
D = HQ * DH
SCALE = 0.08838834764831843
CW = D + 128

RS_HALF = (128, 64, 32, 16)
RS_OFF = (0, 128, 192, 224)
STAGE_ROWS = 240
CHUNK = SQ // N_DEV


def kernel(x, Wq, Wo, K_ext, V_ext):
    skv = K_ext.shape[1]

    xb = x[0].astype(jnp.bfloat16)
    Wqb = Wq.astype(jnp.bfloat16)
    Wob = Wo.astype(jnp.bfloat16)
    Kb = K_ext[0].reshape(skv, D).astype(jnp.bfloat16)
    Vb = V_ext[0].reshape(skv, D).astype(jnp.bfloat16)

    def body(x_ref, wq_ref, wo_ref, k_ref, v_ref, out_ref,
             acc_ref, sstage_ref, rstage_ref, gbuf_ref, q_ref,
             send_sems, recv_sems):
        my = lax.axis_index("i")

        barrier = pltpu.get_barrier_semaphore()
        for k in range(4):
            pl.semaphore_signal(barrier, inc=1, device_id=(my ^ (1 << k),),
                                device_id_type=pl.DeviceIdType.MESH)
        pl.semaphore_wait(barrier, 4)

        q = lax.dot_general(x_ref[...], wq_ref[...],
                            (((1,), (0,)), ((), ())),
                            preferred_element_type=jnp.float32)
        q_ref[...] = (q * SCALE).astype(jnp.bfloat16)

        def attn_half(lo):
            qrows = q_ref[pl.ds(lo, 128), :]
            ls = []
            for h in range(HQ):
                qh = qrows[:, h * DH:(h + 1) * DH]
                s = lax.dot_general(qh, k_ref[:, h * DH:(h + 1) * DH],
                                    (((1,), (1,)), ((), ())),
                                    preferred_element_type=jnp.float32)
                p = jnp.exp(s)
                ls.append(jnp.sum(p, axis=1, keepdims=True))
                acc_h = lax.dot_general(p.astype(jnp.bfloat16),
                                        v_ref[:, h * DH:(h + 1) * DH],
                                        (((1,), (0,)), ((), ())),
                                        preferred_element_type=jnp.float32)
                acc_ref[pl.ds(lo, 128), h * DH:(h + 1) * DH] = acc_h
            lblock = jnp.concatenate(
                ls + [jnp.zeros((128, 128 - HQ), jnp.float32)], axis=1)
            acc_ref[pl.ds(lo, 128), D:] = lblock

        def exchange(step, src_slice, dst_slice, partner):
            return pltpu.make_async_remote_copy(
                src_ref=src_slice,
                dst_ref=dst_slice,
                send_sem=send_sems.at[step],
                recv_sem=recv_sems.at[step],
                device_id=(partner,),
                device_id_type=pl.DeviceIdType.MESH,
            )

        def rs_start(k, slo):
            half = RS_HALF[k]
            sstage_ref[pl.ds(RS_OFF[k], half), :] = (
                acc_ref[pl.ds(slo, half), :].astype(jnp.bfloat16))
            rdma = exchange(k, sstage_ref.at[pl.ds(RS_OFF[k], half)],
                            rstage_ref.at[pl.ds(RS_OFF[k], half)],
                            my ^ (1 << k))
            rdma.start()
            return rdma

        def rs_finish(k, rdma, klo):
            half = RS_HALF[k]
            rdma.wait()
            acc_ref[pl.ds(klo, half), :] = (
                acc_ref[pl.ds(klo, half), :]
                + rstage_ref[pl.ds(RS_OFF[k], half), :].astype(jnp.float32))

        b0 = my & 1
        keep_lo = b0 * 128
        send_lo = (1 - b0) * 128
        attn_half(send_lo)
        rdma0 = rs_start(0, send_lo)
        attn_half(keep_lo)
        rs_finish(0, rdma0, keep_lo)

        cur_lo = keep_lo
        for k in (1, 2, 3):
            half = RS_HALF[k]
            bk = (my >> k) & 1
            klo = cur_lo + bk * half
            slo = cur_lo + (1 - bk) * half
            rdma = rs_start(k, slo)
            rs_finish(k, rdma, klo)
            cur_lo = klo

        a16 = acc_ref[pl.ds(cur_lo, CHUNK), :]
        linv = 1.0 / a16[:, D:D + HQ]
        outs = []
        for h in range(HQ):
            o = a16[:, h * DH:(h + 1) * DH] * linv[:, h:h + 1]
            outs.append(o.astype(jnp.bfloat16))
        o16 = jnp.concatenate(outs, axis=1)
        p16 = lax.dot_general(o16, wo_ref[...],
                              (((1,), (0,)), ((), ())),
                              preferred_element_type=jnp.float32)
        gbuf_ref[pl.ds(cur_lo, CHUNK), :] = p16.astype(jnp.bfloat16)

        sz = CHUNK
        for j, k in enumerate((3, 2, 1, 0)):
            bk = (my >> k) & 1
            rdma = exchange(4 + j, gbuf_ref.at[pl.ds(cur_lo, sz)],
                            gbuf_ref.at[pl.ds(cur_lo, sz)], my ^ (1 << k))
            rdma.start()
            rdma.wait()
            cur_lo = cur_lo - bk * sz
            sz *= 2

        out_ref[0] = gbuf_ref[...].astype(jnp.float32)

    return pl.pallas_call(
        body,
        out_shape=jax.ShapeDtypeStruct((1, SQ, D), jnp.float32),
        in_specs=[pl.BlockSpec(memory_space=pltpu.VMEM)] * 5,
        out_specs=pl.BlockSpec(memory_space=pltpu.VMEM),
        scratch_shapes=[
            pltpu.VMEM((SQ, CW), jnp.float32),
            pltpu.VMEM((STAGE_ROWS, CW), jnp.bfloat16),
            pltpu.VMEM((STAGE_ROWS, CW), jnp.bfloat16),
            pltpu.VMEM((SQ, D), jnp.bfloat16),
            pltpu.VMEM((SQ, D), jnp.bfloat16),
            pltpu.SemaphoreType.DMA((8,)),
            pltpu.SemaphoreType.DMA((8,)),
        ],
        compiler_params=pltpu.CompilerParams(collective_id=0),
    )(xb, Wqb, Wob, Kb, Vb)
